# baseline (device time: 8535 ns/iter reference)
import jax
import jax.numpy as jnp
from jax import lax
from jax.experimental import pallas as pl
from jax.experimental.pallas import tpu as pltpu

N_GLOBAL = 1024.0
EPS = 1e-5
CH = 4


def kernel(x, gamma, beta):
    m, n_local = x.shape
    mo = m // 128
    mo_c = mo // CH
    m_c = m // CH

    def body(x_ref, g_ref, b_ref, out_ref, stats_ref, recv_ref, send_sems, recv_sems):
        my_x = lax.axis_index("x")
        my_y = lax.axis_index("y")
        peer = (my_x, 1 - my_y)

        barrier = pltpu.get_barrier_semaphore()
        pl.semaphore_signal(
            barrier, inc=1, device_id=peer, device_id_type=pl.DeviceIdType.MESH
        )

        g = g_ref[:, :].reshape(1, 1, n_local).astype(jnp.float32)
        b = b_ref[:, :].reshape(1, 1, n_local).astype(jnp.float32)

        rdmas = []
        for c in range(CH):
            xc = (
                x_ref[pl.ds(c * m_c, m_c), :]
                .reshape(mo_c, 128, n_local)
                .astype(jnp.float32)
            )
            stats_ref[c, 0, :, :] = jnp.sum(xc, axis=2)
            stats_ref[c, 1, :, :] = jnp.sum(xc * xc, axis=2)
            if c == 0:
                pl.semaphore_wait(barrier, 1)
            rdma = pltpu.make_async_remote_copy(
                src_ref=stats_ref.at[c],
                dst_ref=recv_ref.at[c],
                send_sem=send_sems.at[c],
                recv_sem=recv_sems.at[c],
                device_id=peer,
                device_id_type=pl.DeviceIdType.MESH,
            )
            rdma.start()
            rdmas.append(rdma)

        for c in range(CH):
            rdmas[c].wait_recv()
            tot = stats_ref[c] + recv_ref[c]
            mean = tot[0] / N_GLOBAL
            var = tot[1] / N_GLOBAL - mean * mean
            inv = lax.rsqrt(var + EPS)
            xc = (
                x_ref[pl.ds(c * m_c, m_c), :]
                .reshape(mo_c, 128, n_local)
                .astype(jnp.float32)
            )
            outc = g * ((xc - mean[:, :, None]) * inv[:, :, None]) + b
            out_ref[pl.ds(c * m_c, m_c), :] = outc.reshape(m_c, n_local).astype(
                out_ref.dtype
            )

        for rdma in rdmas:
            rdma.wait_send()

    return pl.pallas_call(
        body,
        out_shape=jax.ShapeDtypeStruct((m, n_local), jnp.bfloat16),
        in_specs=[
            pl.BlockSpec(memory_space=pltpu.VMEM),
            pl.BlockSpec(memory_space=pltpu.VMEM),
            pl.BlockSpec(memory_space=pltpu.VMEM),
        ],
        out_specs=pl.BlockSpec(memory_space=pltpu.VMEM),
        scratch_shapes=[
            pltpu.VMEM((CH, 2, mo_c, 128), jnp.float32),
            pltpu.VMEM((CH, 2, mo_c, 128), jnp.float32),
            pltpu.SemaphoreType.DMA((CH,)),
            pltpu.SemaphoreType.DMA((CH,)),
        ],
        compiler_params=pltpu.CompilerParams(collective_id=0),
    )(x, gamma.reshape(1, n_local), beta.reshape(1, n_local))


# device time: 8485 ns/iter; 1.0059x vs baseline; 1.0059x over previous
import jax
import jax.numpy as jnp
from jax import lax
from jax.experimental import pallas as pl
from jax.experimental.pallas import tpu as pltpu

N_GLOBAL = 1024.0
EPS = 1e-5
CH = 2


def kernel(x, gamma, beta):
    m, n_local = x.shape
    mo = m // 128
    mo_c = mo // CH
    m_c = m // CH

    def body(x_ref, g_ref, b_ref, out_ref, stats_ref, recv_ref, send_sems, recv_sems):
        my_x = lax.axis_index("x")
        my_y = lax.axis_index("y")
        peer = (my_x, 1 - my_y)

        barrier = pltpu.get_barrier_semaphore()
        pl.semaphore_signal(
            barrier, inc=1, device_id=peer, device_id_type=pl.DeviceIdType.MESH
        )

        g = g_ref[:, :].reshape(1, 1, n_local).astype(jnp.float32)
        b = b_ref[:, :].reshape(1, 1, n_local).astype(jnp.float32)

        rdmas = []
        for c in range(CH):
            xc = (
                x_ref[pl.ds(c * m_c, m_c), :]
                .reshape(mo_c, 128, n_local)
                .astype(jnp.float32)
            )
            stats_ref[c, 0, :, :] = jnp.sum(xc, axis=2)
            stats_ref[c, 1, :, :] = jnp.sum(xc * xc, axis=2)
            if c == 0:
                pl.semaphore_wait(barrier, 1)
            rdma = pltpu.make_async_remote_copy(
                src_ref=stats_ref.at[c],
                dst_ref=recv_ref.at[c],
                send_sem=send_sems.at[c],
                recv_sem=recv_sems.at[c],
                device_id=peer,
                device_id_type=pl.DeviceIdType.MESH,
            )
            rdma.start()
            rdmas.append(rdma)

        for c in range(CH):
            rdmas[c].wait_recv()
            tot = stats_ref[c] + recv_ref[c]
            mean = tot[0] / N_GLOBAL
            var = tot[1] / N_GLOBAL - mean * mean
            inv = lax.rsqrt(var + EPS)
            xc = (
                x_ref[pl.ds(c * m_c, m_c), :]
                .reshape(mo_c, 128, n_local)
                .astype(jnp.float32)
            )
            outc = g * ((xc - mean[:, :, None]) * inv[:, :, None]) + b
            out_ref[pl.ds(c * m_c, m_c), :] = outc.reshape(m_c, n_local).astype(
                out_ref.dtype
            )

        for rdma in rdmas:
            rdma.wait_send()

    return pl.pallas_call(
        body,
        out_shape=jax.ShapeDtypeStruct((m, n_local), jnp.bfloat16),
        in_specs=[
            pl.BlockSpec(memory_space=pltpu.VMEM),
            pl.BlockSpec(memory_space=pltpu.VMEM),
            pl.BlockSpec(memory_space=pltpu.VMEM),
        ],
        out_specs=pl.BlockSpec(memory_space=pltpu.VMEM),
        scratch_shapes=[
            pltpu.VMEM((CH, 2, mo_c, 128), jnp.float32),
            pltpu.VMEM((CH, 2, mo_c, 128), jnp.float32),
            pltpu.SemaphoreType.DMA((CH,)),
            pltpu.SemaphoreType.DMA((CH,)),
        ],
        compiler_params=pltpu.CompilerParams(collective_id=0),
    )(x, gamma.reshape(1, n_local), beta.reshape(1, n_local))
